# baseline (speedup 1.0000x reference)
import functools

import jax
import jax.numpy as jnp
from jax import lax
from jax.experimental import pallas as pl
from jax.experimental.pallas import tpu as pltpu
from jax.experimental.pallas import tpu_sc as plsc

_NUM_CORES = 2
_NUM_SUBCORES = 16
_LANES = 16


def _build_sc_call(n_rows: int, d_model: int, vocab: int):
    num_workers = _NUM_CORES * _NUM_SUBCORES
    n_per_w = n_rows // num_workers
    chunk = 16
    n_chunks = n_per_w // chunk

    mesh = plsc.VectorSubcoreMesh(
        core_axis_name="c",
        subcore_axis_name="s",
        num_cores=_NUM_CORES,
        num_subcores=_NUM_SUBCORES,
    )

    @functools.partial(
        pl.kernel,
        out_type=jax.ShapeDtypeStruct((n_rows, d_model), jnp.float32),
        mesh=mesh,
        scratch_types=[
            pltpu.VMEM((n_per_w,), jnp.int32),
            pltpu.VMEM((chunk, d_model), jnp.float32),
            pltpu.VMEM((chunk, d_model), jnp.float32),
            pltpu.SemaphoreType.DMA,
        ],
    )
    def sc_add_pe(x_hbm, idx_hbm, pe_hbm, out_hbm, idx_v, pe_v, x_v, gsem):
        cid = lax.axis_index("c")
        sid = lax.axis_index("s")
        wid = sid * _NUM_CORES + cid
        base = wid * n_per_w

        pltpu.sync_copy(idx_hbm.at[pl.ds(base, n_per_w)], idx_v)

        def chunk_body(g, carry):
            off = g * chunk
            gather = pltpu.async_copy(
                pe_hbm.at[idx_v.at[pl.ds(off, chunk)]], pe_v, gsem
            )
            pltpu.sync_copy(x_hbm.at[pl.ds(base + off, chunk)], x_v)
            gather.wait()

            def row_body(r, c2):
                for j in range(d_model // _LANES):
                    sl = pl.ds(j * _LANES, _LANES)
                    x_v[r, sl] = x_v[r, sl] + pe_v[r, sl]
                return c2

            lax.fori_loop(0, chunk, row_body, 0, unroll=False)
            pltpu.sync_copy(x_v, out_hbm.at[pl.ds(base + off, chunk)])
            return carry

        lax.fori_loop(0, n_chunks, chunk_body, 0, unroll=False)

    return sc_add_pe


def kernel(x, rand_idx, pe):
    b, s, d = x.shape
    n_rows = b * s
    vocab = pe.shape[1]

    x_flat = x.reshape(n_rows, d)
    idx_flat = rand_idx.reshape(n_rows).astype(jnp.int32)
    pe_flat = pe.reshape(vocab, d)

    out = _build_sc_call(n_rows, d, vocab)(x_flat, idx_flat, pe_flat)
    return out.reshape(1, b, s, d)
